# Initial kernel scaffold; baseline (speedup 1.0000x reference)
import jax
import jax.numpy as jnp
from jax.experimental import pallas as pl


def kernel(x, fp_values):
    raise NotImplementedError("write your pallas kernel here")


# speedup vs baseline: 15709.6878x; 15709.6878x over previous
import jax
import jax.numpy as jnp
from jax.experimental import pallas as pl
from jax.experimental.pallas import tpu as pltpu


def _quant_body(v_ref, x_ref, o_ref):
    a0, a1, a2, a3 = v_ref[0], v_ref[1], v_ref[2], v_ref[3]
    b0, b1 = jnp.minimum(a0, a1), jnp.maximum(a0, a1)
    b2, b3 = jnp.minimum(a2, a3), jnp.maximum(a2, a3)
    c0, c2 = jnp.minimum(b0, b2), jnp.maximum(b0, b2)
    c1, c3 = jnp.minimum(b1, b3), jnp.maximum(b1, b3)
    v0, v3 = c0, c3
    v1, v2 = jnp.minimum(c1, c2), jnp.maximum(c1, c2)

    x = x_ref[...]
    c_hi = x > v2
    c_mid = x > v1
    low = jnp.where(c_hi, v2, jnp.where(c_mid, v1, v0))
    high = jnp.where(c_hi, v3, jnp.where(c_mid, v2, v1))
    o_ref[...] = jnp.where(jnp.abs(x - low) <= jnp.abs(x - high), low, high)


def kernel(x, fp_values):
    orig_shape = x.shape
    x2 = x.reshape(-1, orig_shape[-1])
    rows, cols = x2.shape
    block_rows = 256
    grid = (rows // block_rows,)
    out = pl.pallas_call(
        _quant_body,
        grid=grid,
        in_specs=[
            pl.BlockSpec(memory_space=pltpu.SMEM),
            pl.BlockSpec((block_rows, cols), lambda i: (i, 0)),
        ],
        out_specs=pl.BlockSpec((block_rows, cols), lambda i: (i, 0)),
        out_shape=jax.ShapeDtypeStruct((rows, cols), x.dtype),
    )(fp_values, x2)
    return out.reshape(orig_shape)
